# Initial kernel scaffold; baseline (speedup 1.0000x reference)
import jax
import jax.numpy as jnp
from jax.experimental import pallas as pl


def kernel(x, u, params, batch, edge_index):
    raise NotImplementedError("write your pallas kernel here")


# speedup vs baseline: 5.7003x; 5.7003x over previous
import functools
import math

import jax
import jax.numpy as jnp
from jax import lax
from jax.experimental import pallas as pl
from jax.experimental.pallas import tpu as pltpu
from jax.experimental.pallas import tpu_sc as plsc

_EPS = 1e-5
_NGRAPH = 64
_K = 16


def _sc_gather(table, idx):
  M = idx.shape[0]
  C = table.shape[1]
  info = plsc.get_sparse_core_info()
  nc, ns = info.num_cores, info.num_subcores
  nw = nc * ns
  rows_per_w = M // nw
  ch = 128
  n_chunks = rows_per_w // ch
  mesh = plsc.VectorSubcoreMesh(core_axis_name="c", subcore_axis_name="s")

  @functools.partial(
      pl.kernel,
      out_type=jax.ShapeDtypeStruct((M, C), jnp.float32),
      mesh=mesh,
      scratch_types=[
          pltpu.VMEM((ch,), jnp.int32),
          pltpu.VMEM((ch, C), jnp.float32),
          pltpu.SemaphoreType.DMA,
      ],
      compiler_params=pltpu.CompilerParams(use_tc_tiling_on_sc=False),
  )
  def k(table_hbm, idx_hbm, out_hbm, idx_v, rows_v, sem):
    wid = lax.axis_index("s") * nc + lax.axis_index("c")
    base = wid * rows_per_w

    def body(i, carry):
      off = pl.multiple_of(base + i * ch, 8)
      pltpu.sync_copy(idx_hbm.at[pl.ds(off, ch)], idx_v)
      pltpu.async_copy(table_hbm.at[idx_v], rows_v, sem).wait()
      pltpu.sync_copy(rows_v, out_hbm.at[pl.ds(off, ch)])
      return carry

    lax.fori_loop(0, n_chunks, body, 0)

  return k(table, idx)


def _bn0_body(x_ref, g_ref, be_ref, o_ref):
  x = x_ref[...]
  m = jnp.mean(x, axis=0, keepdims=True)
  v = jnp.mean((x - m) * (x - m), axis=0, keepdims=True)
  o_ref[...] = (x - m) * lax.rsqrt(v + _EPS) * g_ref[...] + be_ref[...]


def _knn_body(xb_ref, xt_ref, bc_ref, br_ref, idx_ref):
  xb = xb_ref[...]
  xt = xt_ref[...]
  mm = jnp.dot(xb, xt, preferred_element_type=jnp.float32,
               precision=lax.Precision.DEFAULT)
  x2r = jnp.sum(xb * xb, axis=1, keepdims=True)
  x2c = jnp.sum(xt * xt, axis=0, keepdims=True)
  d = x2r + x2c - 2.0 * mm
  same = bc_ref[...] == br_ref[...]
  d = jnp.where(same, d, jnp.inf)
  colid = lax.broadcasted_iota(jnp.int32, d.shape, 1)
  big = jnp.int32(2 ** 30)
  for j in range(_K):
    m = jnp.min(d, axis=1, keepdims=True)
    c = jnp.min(jnp.where(d == m, colid, big), axis=1, keepdims=True)
    idx_ref[:, j:j + 1] = c
    d = jnp.where(colid == c, jnp.inf, d)


def _linear2_body(x_ref, wa_ref, wb_ref, b_ref, p_ref, q_ref):
  x = x_ref[...]
  p_ref[...] = jnp.dot(x, wa_ref[...], preferred_element_type=jnp.float32,
                       precision=lax.Precision.DEFAULT) + b_ref[...]
  q_ref[...] = jnp.dot(x, wb_ref[...], preferred_element_type=jnp.float32,
                       precision=lax.Precision.DEFAULT)


def _acc_stats(st_ref, h):
  s1 = jnp.sum(h, axis=0, keepdims=True)
  s2 = jnp.sum(h * h, axis=0, keepdims=True)
  st = jnp.concatenate([s1, s2], axis=0)

  @pl.when(pl.program_id(0) == 0)
  def _():
    st_ref[...] = st

  @pl.when(pl.program_id(0) > 0)
  def _():
    st_ref[...] += st


def _pnode_body(x_ref, w_ref, b_ref, p_ref):
  p_ref[...] = jnp.dot(x_ref[...], w_ref[...],
                       preferred_element_type=jnp.float32,
                       precision=lax.Precision.DEFAULT) + b_ref[...]


def _h1_conv_body(p_ref, xn_ref, xj_ref, wb_ref, rep_ref, h_ref, st_ref):
  rep = rep_ref[...]
  xi = jnp.dot(rep, xn_ref[...], preferred_element_type=jnp.float32,
               precision=lax.Precision.HIGHEST)
  t = xj_ref[...] - xi
  h = jnp.dot(rep, p_ref[...], preferred_element_type=jnp.float32,
              precision=lax.Precision.HIGHEST)
  h = h + jnp.dot(t, wb_ref[...], preferred_element_type=jnp.float32,
                  precision=lax.Precision.DEFAULT)
  h_ref[...] = h
  _acc_stats(st_ref, h)


def _h1_pair_body(pg_ref, qg_ref, h_ref, st_ref):
  h = pg_ref[...] + qg_ref[...]
  h_ref[...] = h
  _acc_stats(st_ref, h)


def _flin_body(h_ref, s_ref, t_ref, w_ref, b_ref, o_ref, st_ref):
  x = jnp.maximum(h_ref[...] * s_ref[...] + t_ref[...], 0.0)
  hn = jnp.dot(x, w_ref[...], preferred_element_type=jnp.float32,
               precision=lax.Precision.DEFAULT) + b_ref[...]
  o_ref[...] = hn
  _acc_stats(st_ref, hn)


def _finalize_body(h_ref, s_ref, t_ref, pm_ref, o_ref):
  x = jnp.maximum(h_ref[...] * s_ref[...] + t_ref[...], 0.0)
  o_ref[...] = jnp.dot(pm_ref[...], x, preferred_element_type=jnp.float32,
                       precision=lax.Precision.HIGHEST)


def _final_body(h_ref, s_ref, t_ref, w_ref, b_ref, tr_ref, tc_ref, seg_ref):
  x = jnp.maximum(h_ref[...] * s_ref[...] + t_ref[...], 0.0)
  f = jnp.dot(x, w_ref[...], preferred_element_type=jnp.float32,
              precision=lax.Precision.DEFAULT) + b_ref[...]
  tr = tr_ref[...]
  tc = tc_ref[...]
  dy = tr[:, 1:2] - tc[:, 1:2]
  a = tr[:, 2:3] - tc[:, 2:3] + math.pi
  dphi = lax.rem(a, jnp.float32(2.0 * math.pi)) - math.pi
  dr = jnp.sqrt(dy * dy + dphi * dphi)
  val = dr * f
  bat = lax.bitcast_convert_type(tr[:, 3:4], jnp.int32)
  gid = lax.broadcasted_iota(jnp.int32, (1, _NGRAPH), 1)
  contrib = jnp.sum(jnp.where(bat == gid, val, 0.0), axis=0, keepdims=True)

  @pl.when(pl.program_id(0) == 0)
  def _():
    seg_ref[...] = contrib

  @pl.when(pl.program_id(0) > 0)
  def _():
    seg_ref[...] += contrib


def _bn0_call(x, g, be):
  n, d = x.shape
  return pl.pallas_call(
      _bn0_body,
      out_shape=jax.ShapeDtypeStruct((n, d), jnp.float32),
  )(x, g.reshape(1, -1), be.reshape(1, -1))


def _knn_call(xpad, xpadt, bcol, brow):
  n = xpad.shape[0]
  rt = 128
  return pl.pallas_call(
      _knn_body,
      grid=(n // rt,),
      in_specs=[
          pl.BlockSpec((rt, 128), lambda i: (i, 0)),
          pl.BlockSpec((128, n), lambda i: (0, 0)),
          pl.BlockSpec((rt, 1), lambda i: (i, 0)),
          pl.BlockSpec((1, n), lambda i: (0, 0)),
      ],
      out_specs=pl.BlockSpec((rt, _K), lambda i: (i, 0)),
      out_shape=jax.ShapeDtypeStruct((n, _K), jnp.int32),
  )(xpad, xpadt, bcol, brow)


def _linear2_call(xin, wa, wb, b1):
  n, d = xin.shape
  c = wa.shape[1]
  rt = 1024
  return pl.pallas_call(
      _linear2_body,
      grid=(n // rt,),
      in_specs=[
          pl.BlockSpec((rt, d), lambda i: (i, 0)),
          pl.BlockSpec((d, c), lambda i: (0, 0)),
          pl.BlockSpec((d, c), lambda i: (0, 0)),
          pl.BlockSpec((1, c), lambda i: (0, 0)),
      ],
      out_specs=[
          pl.BlockSpec((rt, c), lambda i: (i, 0)),
          pl.BlockSpec((rt, c), lambda i: (i, 0)),
      ],
      out_shape=[
          jax.ShapeDtypeStruct((n, c), jnp.float32),
          jax.ShapeDtypeStruct((n, c), jnp.float32),
      ],
  )(xin, wa, wb, b1.reshape(1, -1))


def _stats_outs(nk, c, rt):
  return dict(
      out_specs=[
          pl.BlockSpec((rt, c), lambda i: (i, 0)),
          pl.BlockSpec((2, c), lambda i: (0, 0)),
      ],
      out_shape=[
          jax.ShapeDtypeStruct((nk, c), jnp.float32),
          jax.ShapeDtypeStruct((2, c), jnp.float32),
      ],
  )


def _pnode_call(xin, w, b):
  n, d = xin.shape
  c = w.shape[1]
  rt = 1024
  return pl.pallas_call(
      _pnode_body,
      grid=(n // rt,),
      in_specs=[
          pl.BlockSpec((rt, d), lambda i: (i, 0)),
          pl.BlockSpec((d, c), lambda i: (0, 0)),
          pl.BlockSpec((1, c), lambda i: (0, 0)),
      ],
      out_specs=pl.BlockSpec((rt, c), lambda i: (i, 0)),
      out_shape=jax.ShapeDtypeStruct((n, c), jnp.float32),
  )(xin, w, b.reshape(1, -1))


def _h1_conv_call(p, xnpad, xjg, wbpad, rep):
  nk, dpad = xjg.shape
  c = p.shape[1]
  rt = 2048
  return pl.pallas_call(
      _h1_conv_body,
      grid=(nk // rt,),
      in_specs=[
          pl.BlockSpec((128, c), lambda i: (i, 0)),
          pl.BlockSpec((128, dpad), lambda i: (i, 0)),
          pl.BlockSpec((rt, dpad), lambda i: (i, 0)),
          pl.BlockSpec((dpad, c), lambda i: (0, 0)),
          pl.BlockSpec((rt, 128), lambda i: (0, 0)),
      ],
      **_stats_outs(nk, c, rt),
  )(p, xnpad, xjg, wbpad, rep)


def _h1_pair_call(pg, qg):
  nk, c = pg.shape
  rt = 2048
  return pl.pallas_call(
      _h1_pair_body,
      grid=(nk // rt,),
      in_specs=[
          pl.BlockSpec((rt, c), lambda i: (i, 0)),
          pl.BlockSpec((rt, c), lambda i: (i, 0)),
      ],
      **_stats_outs(nk, c, rt),
  )(pg, qg)


def _flin_call(h, s, t, w, b):
  nk, din = h.shape
  c = w.shape[1]
  rt = 2048
  return pl.pallas_call(
      _flin_body,
      grid=(nk // rt,),
      in_specs=[
          pl.BlockSpec((rt, din), lambda i: (i, 0)),
          pl.BlockSpec((1, din), lambda i: (0, 0)),
          pl.BlockSpec((1, din), lambda i: (0, 0)),
          pl.BlockSpec((din, c), lambda i: (0, 0)),
          pl.BlockSpec((1, c), lambda i: (0, 0)),
      ],
      **_stats_outs(nk, c, rt),
  )(h, s, t, w, b.reshape(1, -1))


def _finalize_call(h3, s, t, pm):
  nk, c = h3.shape
  rt = 2048
  n = nk // _K
  return pl.pallas_call(
      _finalize_body,
      grid=(nk // rt,),
      in_specs=[
          pl.BlockSpec((rt, c), lambda i: (i, 0)),
          pl.BlockSpec((1, c), lambda i: (0, 0)),
          pl.BlockSpec((1, c), lambda i: (0, 0)),
          pl.BlockSpec((rt // _K, rt), lambda i: (0, 0)),
      ],
      out_specs=pl.BlockSpec((rt // _K, c), lambda i: (i, 0)),
      out_shape=jax.ShapeDtypeStruct((n, c), jnp.float32),
  )(h3, s, t, pm)


def _final_call(h2, s, t, w3, b3, tg_row, tg_col):
  e, din = h2.shape
  rt = 2048
  return pl.pallas_call(
      _final_body,
      grid=(e // rt,),
      in_specs=[
          pl.BlockSpec((rt, din), lambda i: (i, 0)),
          pl.BlockSpec((1, din), lambda i: (0, 0)),
          pl.BlockSpec((1, din), lambda i: (0, 0)),
          pl.BlockSpec((din, 1), lambda i: (0, 0)),
          pl.BlockSpec((1, 1), lambda i: (0, 0)),
          pl.BlockSpec((rt, 16), lambda i: (i, 0)),
          pl.BlockSpec((rt, 16), lambda i: (i, 0)),
      ],
      out_specs=pl.BlockSpec((1, _NGRAPH), lambda i: (0, 0)),
      out_shape=jax.ShapeDtypeStruct((1, _NGRAPH), jnp.float32),
  )(h2, s, t, w3, b3.reshape(1, 1), tg_row, tg_col)


def _st_from_stats(stats, g, be, n_rows):
  mean = stats[0] / n_rows
  var = jnp.maximum(stats[1] / n_rows - mean * mean, 0.0)
  s = g * lax.rsqrt(var + _EPS)
  t = be - mean * s
  return s.reshape(1, -1), t.reshape(1, -1)


def _edge_conv(layers, xin, bcol, brow, rep, pm):
  n, d = xin.shape
  xpad = jnp.pad(xin, ((0, 0), (0, 128 - d)))
  idx = _knn_call(xpad, xpad.T, bcol, brow)
  w1 = layers[0]['W']
  dpad = ((d + 15) // 16) * 16
  xpad_d = xpad[:, :dpad]
  p = _pnode_call(xin, w1[:d], layers[0]['b'])
  xjg = _sc_gather(xpad_d, idx.reshape(-1))
  wbpad = jnp.pad(w1[d:], ((0, dpad - d), (0, 0)))
  h1, st1 = _h1_conv_call(p, xpad_d, xjg, wbpad, rep)
  nk = h1.shape[0]
  s1, t1 = _st_from_stats(st1, layers[0]['g'], layers[0]['be'], nk)
  h2, st2 = _flin_call(h1, s1, t1, layers[1]['W'], layers[1]['b'])
  s2, t2 = _st_from_stats(st2, layers[1]['g'], layers[1]['be'], nk)
  h3, st3 = _flin_call(h2, s2, t2, layers[2]['W'], layers[2]['b'])
  s3, t3 = _st_from_stats(st3, layers[2]['g'], layers[2]['be'], nk)
  return _finalize_call(h3, s3, t3, pm)


def kernel(x, u, params, batch, edge_index):
  n = x.shape[0]
  bcol = batch.reshape(n, 1)
  brow = batch.reshape(1, n)
  rep = jnp.kron(jnp.eye(128, dtype=jnp.float32),
                 jnp.ones((_K, 1), jnp.float32))
  pm = rep.T / jnp.float32(_K)

  x1 = _bn0_call(x, params['bn0']['g'], params['bn0']['be'])

  xc = x
  for name in ('conv1', 'conv2', 'conv3'):
    out = _edge_conv(params[name], xc, bcol, brow, rep, pm)
    xc = jnp.concatenate([x1, out], axis=1)

  row, col = edge_index[0], edge_index[1]
  olayers = params['out']
  dxc = xc.shape[1]
  w1 = olayers[0]['W']
  pnode, qnode = _linear2_call(xc, w1[:dxc], w1[dxc:], olayers[0]['b'])
  pg = _sc_gather(pnode, row)
  qg = _sc_gather(qnode, col)
  h1, st1 = _h1_pair_call(pg, qg)
  e = h1.shape[0]
  s1, t1 = _st_from_stats(st1, olayers[0]['g'], olayers[0]['be'], e)
  h2, st2 = _flin_call(h1, s1, t1, olayers[1]['W'], olayers[1]['b'])
  s2, t2 = _st_from_stats(st2, olayers[1]['g'], olayers[1]['be'], e)

  tnode = jnp.zeros((n, 16), jnp.float32)
  tnode = tnode.at[:, :3].set(x)
  tnode = tnode.at[:, 3].set(lax.bitcast_convert_type(batch, jnp.float32))
  tg_row = _sc_gather(tnode, row)
  tg_col = _sc_gather(tnode, col)

  seg = _final_call(h2, s2, t2, olayers[2]['W'], olayers[2]['b'],
                    tg_row, tg_col)
  emd = seg[0] + jnp.abs(u[:, 0] - u[:, 1])
  return emd[:, None]
